# baseline (speedup 1.0000x reference)
import jax
import jax.numpy as jnp
from jax.experimental import pallas as pl


def _add_kernel(x_ref, pos_ref, o_ref):
    o_ref[...] = x_ref[...] + pos_ref[...]


def kernel(x, positions):
    B, S, D = x.shape
    pos = positions[0, :S, :]
    x2 = x.reshape(B * S, D)

    BS = 2048
    n_halves = S // BS

    def xmap(i, B=B, n=n_halves):
        return ((i % B) * n + i // B, 0)

    def pmap(i, B=B):
        return (i // B, 0)

    out = pl.pallas_call(
        _add_kernel,
        grid=(B * n_halves,),
        in_specs=[
            pl.BlockSpec((BS, D), xmap),
            pl.BlockSpec((BS, D), pmap),
        ],
        out_specs=pl.BlockSpec((BS, D), xmap),
        out_shape=jax.ShapeDtypeStruct((B * S, D), x.dtype),
    )(x2, pos)
    return out.reshape(B, S, D)
